# baseline (speedup 1.0000x reference)
import functools

import jax
import jax.numpy as jnp
from jax import lax
from jax.experimental import pallas as pl
from jax.experimental.pallas import tpu as pltpu
from jax.experimental.pallas import tpu_sc as plsc

V = 201000
V_PAD = 201088
D = 128
NF = 3
B = 4096
R = B * NF

_NC, _NS = 2, 16
NW = _NC * _NS
B_PER_W = R // NW


@functools.cache
def _make_sc_gather():
    mesh = plsc.VectorSubcoreMesh(core_axis_name="c", subcore_axis_name="s",
                                  num_cores=_NC, num_subcores=_NS)

    @functools.partial(
        pl.kernel,
        out_type=(
            jax.ShapeDtypeStruct((R, D), jnp.float32),
            jax.ShapeDtypeStruct((R,), jnp.float32),
        ),
        mesh=mesh,
        scratch_types=(
            pltpu.VMEM((B_PER_W,), jnp.int32),
            pltpu.VMEM((B_PER_W, D), jnp.float32),
            pltpu.VMEM((B_PER_W,), jnp.float32),
            pltpu.SemaphoreType.DMA,
            pltpu.SemaphoreType.DMA,
        ),
    )
    def _sc_gather(emb_hbm, fc_hbm, idx_hbm, emb_out, fc_out,
                   idx_v, rows_v, fc_v, sem_e, sem_f):
        wid = lax.axis_index("s") * _NC + lax.axis_index("c")
        base = wid * B_PER_W
        pltpu.sync_copy(idx_hbm.at[pl.ds(base, B_PER_W)], idx_v)
        cp_e = pltpu.async_copy(emb_hbm.at[idx_v], rows_v, sem_e)
        cp_f = pltpu.async_copy(fc_hbm.at[idx_v], fc_v, sem_f)
        cp_e.wait()
        pltpu.sync_copy(rows_v, emb_out.at[pl.ds(base, B_PER_W)])
        cp_f.wait()
        pltpu.sync_copy(fc_v, fc_out.at[pl.ds(base, B_PER_W)])

    return _sc_gather


_BLK = 1024
_GRID = B // _BLK


def _dense_body(e0_ref, e1_ref, e2_ref, fc0_ref, fc1_ref, fc2_ref, bias_ref,
                w0_ref, b0_ref, w1_ref, b1_ref, w2_ref, b2_ref, w3_ref,
                b3_ref, out_ref):
    e0 = e0_ref[...]
    e1 = e1_ref[...]
    e2 = e2_ref[...]
    q = e0 * e1 + (e0 + e1) * e2
    bf = jnp.bfloat16
    w0 = w0_ref[...]
    a = jnp.dot(e0.astype(bf), w0[:D].astype(bf),
                preferred_element_type=jnp.float32)
    a += jnp.dot(e1.astype(bf), w0[D:2 * D].astype(bf),
                 preferred_element_type=jnp.float32)
    a += jnp.dot(e2.astype(bf), w0[2 * D:].astype(bf),
                 preferred_element_type=jnp.float32)
    a = jnp.maximum(a + b0_ref[...], 0.0)
    a = jnp.maximum(jnp.dot(a.astype(bf), w1_ref[...].astype(bf),
                            preferred_element_type=jnp.float32) + b1_ref[...], 0.0)
    a = jnp.maximum(jnp.dot(a.astype(bf), w2_ref[...].astype(bf),
                            preferred_element_type=jnp.float32) + b2_ref[...], 0.0)
    mlp = jnp.dot(a.astype(bf), w3_ref[...].astype(bf),
                  preferred_element_type=jnp.float32)
    fm2 = jnp.dot(q, jnp.full((D, 1), 1.0, jnp.float32),
                  preferred_element_type=jnp.float32)
    z = (mlp[:, 0] + fm2[:, 0] + fc0_ref[...] + fc1_ref[...]
         + fc2_ref[...] + bias_ref[0] + b3_ref[0])
    out_ref[...] = jax.nn.sigmoid(z)


def _dense(emb_rows, fc_vals, bias, W0, b0, W1, b1, W2, b2, W3, b3):
    full = lambda shape: pl.BlockSpec(shape, lambda i: (0,) * len(shape))
    eblk = lambda f: pl.BlockSpec((_BLK, D), lambda i, f=f: (f * _GRID + i, 0))
    fcblk = lambda f: pl.BlockSpec((_BLK,), lambda i, f=f: (f * _GRID + i,))
    return pl.pallas_call(
        _dense_body,
        grid=(_GRID,),
        in_specs=[
            eblk(0), eblk(1), eblk(2),
            fcblk(0), fcblk(1), fcblk(2),
            full((1,)),
            full((NF * D, 256)), full((256,)),
            full((256, 128)), full((128,)),
            full((128, 64)), full((64,)),
            full((64, 1)), full((1,)),
        ],
        out_specs=pl.BlockSpec((_BLK,), lambda i: (i,)),
        out_shape=jax.ShapeDtypeStruct((B,), jnp.float32),
    )(emb_rows, emb_rows, emb_rows, fc_vals, fc_vals, fc_vals,
      bias, W0, b0, W1, b1, W2, b2, W3, b3)


def kernel(x, bias, fc_table, emb_table, W0, b0, W1, b1, W2, b2, W3, b3):
    idx = x.T.reshape(-1).astype(jnp.int32)
    fc_flat = jnp.pad(fc_table, ((0, V_PAD - V), (0, 0))).reshape(V_PAD)
    emb_rows, fc_vals = _make_sc_gather()(emb_table, fc_flat, idx)
    return _dense(emb_rows, fc_vals, bias, W0, b0, W1, b1, W2, b2, W3, b3)


# speedup vs baseline: 1.0894x; 1.0894x over previous
import functools

import jax
import jax.numpy as jnp
from jax import lax
from jax.experimental import pallas as pl
from jax.experimental.pallas import tpu as pltpu
from jax.experimental.pallas import tpu_sc as plsc

V = 201000
V_PAD = 201088
D = 128
NF = 3
B = 4096
R = B * NF

_NC, _NS = 2, 16
NW = _NC * _NS
B_PER_W = R // NW


@functools.cache
def _make_sc_gather():
    mesh = plsc.VectorSubcoreMesh(core_axis_name="c", subcore_axis_name="s",
                                  num_cores=_NC, num_subcores=_NS)

    @functools.partial(
        pl.kernel,
        out_type=jax.ShapeDtypeStruct((R, D), jnp.float32),
        mesh=mesh,
        scratch_types=(
            pltpu.VMEM((B_PER_W,), jnp.int32),
            pltpu.VMEM((B_PER_W, D), jnp.float32),
            pltpu.SemaphoreType.DMA,
        ),
    )
    def _sc_gather(emb_hbm, idx_hbm, emb_out, idx_v, rows_v, sem_e):
        wid = lax.axis_index("s") * _NC + lax.axis_index("c")
        base = wid * B_PER_W
        pltpu.sync_copy(idx_hbm.at[pl.ds(base, B_PER_W)], idx_v)
        pltpu.async_copy(emb_hbm.at[idx_v], rows_v, sem_e).wait()
        pltpu.sync_copy(rows_v, emb_out.at[pl.ds(base, B_PER_W)])

    return _sc_gather


@functools.cache
def _make_sc_fc_gather():
    mesh = plsc.VectorSubcoreMesh(core_axis_name="c", subcore_axis_name="s",
                                  num_cores=_NC, num_subcores=_NS)

    @functools.partial(
        pl.kernel,
        out_type=jax.ShapeDtypeStruct((R,), jnp.float32),
        mesh=mesh,
        scratch_types=(
            pltpu.VMEM((B_PER_W,), jnp.int32),
            pltpu.VMEM((B_PER_W,), jnp.float32),
            pltpu.SemaphoreType.DMA,
        ),
    )
    def _sc_fc(fc_hbm, idx_hbm, fc_out, idx_v, fc_v, sem_f):
        wid = lax.axis_index("s") * _NC + lax.axis_index("c")
        base = wid * B_PER_W
        pltpu.sync_copy(idx_hbm.at[pl.ds(base, B_PER_W)], idx_v)
        pltpu.async_copy(fc_hbm.at[idx_v], fc_v, sem_f).wait()
        pltpu.sync_copy(fc_v, fc_out.at[pl.ds(base, B_PER_W)])

    return _sc_fc


_BLK = 1024
_GRID = B // _BLK


def _dense_body(e0_ref, e1_ref, e2_ref, fc0_ref, fc1_ref, fc2_ref, bias_ref,
                w0_ref, b0_ref, w1_ref, b1_ref, w2_ref, b2_ref, w3_ref,
                b3_ref, out_ref):
    e0 = e0_ref[...]
    e1 = e1_ref[...]
    e2 = e2_ref[...]
    q = e0 * e1 + (e0 + e1) * e2
    bf = jnp.bfloat16
    w0 = w0_ref[...]
    a = jnp.dot(e0.astype(bf), w0[:D].astype(bf),
                preferred_element_type=jnp.float32)
    a += jnp.dot(e1.astype(bf), w0[D:2 * D].astype(bf),
                 preferred_element_type=jnp.float32)
    a += jnp.dot(e2.astype(bf), w0[2 * D:].astype(bf),
                 preferred_element_type=jnp.float32)
    a = jnp.maximum(a + b0_ref[...], 0.0)
    a = jnp.maximum(jnp.dot(a.astype(bf), w1_ref[...].astype(bf),
                            preferred_element_type=jnp.float32) + b1_ref[...], 0.0)
    a = jnp.maximum(jnp.dot(a.astype(bf), w2_ref[...].astype(bf),
                            preferred_element_type=jnp.float32) + b2_ref[...], 0.0)
    mlp = jnp.dot(a.astype(bf), w3_ref[...].astype(bf),
                  preferred_element_type=jnp.float32)
    fm2 = jnp.dot(q, jnp.full((D, 1), 1.0, jnp.float32),
                  preferred_element_type=jnp.float32)
    z = (mlp[:, 0] + fm2[:, 0] + fc0_ref[...] + fc1_ref[...]
         + fc2_ref[...] + bias_ref[0] + b3_ref[0])
    out_ref[...] = jax.nn.sigmoid(z)


def _dense(emb_rows, fc_vals, bias, W0, b0, W1, b1, W2, b2, W3, b3):
    full = lambda shape: pl.BlockSpec(shape, lambda i: (0,) * len(shape))
    eblk = lambda f: pl.BlockSpec((_BLK, D), lambda i, f=f: (f * _GRID + i, 0))
    fcblk = lambda f: pl.BlockSpec((_BLK,), lambda i, f=f: (f * _GRID + i,))
    return pl.pallas_call(
        _dense_body,
        grid=(_GRID,),
        in_specs=[
            eblk(0), eblk(1), eblk(2),
            fcblk(0), fcblk(1), fcblk(2),
            full((1,)),
            full((NF * D, 256)), full((256,)),
            full((256, 128)), full((128,)),
            full((128, 64)), full((64,)),
            full((64, 1)), full((1,)),
        ],
        out_specs=pl.BlockSpec((_BLK,), lambda i: (i,)),
        out_shape=jax.ShapeDtypeStruct((B,), jnp.float32),
    )(emb_rows, emb_rows, emb_rows, fc_vals, fc_vals, fc_vals,
      bias, W0, b0, W1, b1, W2, b2, W3, b3)


def kernel(x, bias, fc_table, emb_table, W0, b0, W1, b1, W2, b2, W3, b3):
    idx = x.T.reshape(-1).astype(jnp.int32)
    fc_flat = jnp.pad(fc_table, ((0, V_PAD - V), (0, 0))).reshape(V_PAD)
    emb_rows = _make_sc_gather()(emb_table, idx)
    idx_after = lax.optimization_barrier((idx, emb_rows))[0]
    fc_vals = _make_sc_fc_gather()(fc_flat, idx_after)
    return _dense(emb_rows, fc_vals, bias, W0, b0, W1, b1, W2, b2, W3, b3)
